# baseline (speedup 1.0000x reference)
import functools

import jax
import jax.numpy as jnp
from jax.experimental import pallas as pl
from jax.experimental.pallas import tpu as pltpu

_BB = 16


def _dot(a, b):
    return jnp.dot(a, b, preferred_element_type=jnp.float32)


def _bdot(a, b):
    return jnp.dot(a.astype(jnp.bfloat16), b.astype(jnp.bfloat16),
                   preferred_element_type=jnp.float32)


def _gnn_body(si_ref, sw_ref, x_ref, emb_ref, Us_ref, Vs_ref, Aws_ref,
              Bws_ref, Cws_ref, x_out_ref, sw_out_ref,
              rt_ref, p_ref, pu_ref, pv_ref, sm_ref, tr_ref, er_ref, wbd_ref,
              *, V, H, L, NSW, BB):
    VP = (V + 7) // 8 * 8
    VVP = V * VP
    H2 = 2 * H
    f32 = jnp.float32
    bf16 = jnp.bfloat16

    @pl.when(pl.program_id(0) == 0)
    def _precompute():
        r_iota = jax.lax.broadcasted_iota(jnp.int32, (VVP, 1), 0)
        u_idx = r_iota // VP
        v_idx = r_iota - u_idx * VP
        valid = v_idx < V

        rrow = jax.lax.broadcasted_iota(jnp.int32, (V, VVP), 0)
        rcol = jax.lax.broadcasted_iota(jnp.int32, (V, VVP), 1)
        rt_ref[...] = ((rrow == rcol // VP) & (rcol % VP < V)).astype(bf16)

        a0 = jax.lax.broadcasted_iota(jnp.int32, (V, V), 0)
        a1 = jax.lax.broadcasted_iota(jnp.int32, (V, V), 1)
        cls = jnp.zeros((VVP, 1), dtype=jnp.bool_)
        Sm = jnp.zeros((V, V), dtype=jnp.bool_)
        for k in range(NSW):
            i0 = si_ref[0, k]
            i1 = si_ref[1, k]
            cls = cls | ((u_idx == i0) & (v_idx == i1)) | ((u_idx == i1) & (v_idx == i0))
            Sm = Sm | ((a0 == i0) & (a1 == i1)) | ((a0 == i1) & (a1 == i0))
        cf = (cls & valid).astype(f32)
        sm_ref[...] = Sm.astype(f32)

        c1 = jax.lax.broadcasted_iota(jnp.int32, (1, VVP), 1)
        cv = jax.lax.broadcasted_iota(jnp.int32, (1, V), 1)
        prows, urows, vrows = [], [], []
        for k in range(NSW):
            iu = sw_ref[0, k]
            ju = sw_ref[1, k]
            prows.append((c1 == iu * VP + ju).astype(f32))
            urows.append((cv == iu).astype(f32))
            vrows.append((cv == ju).astype(f32))
        p_ref[...] = jnp.concatenate(prows, axis=0).astype(bf16)
        pu_ref[...] = jnp.concatenate(urows, axis=0)
        pv_ref[...] = jnp.concatenate(vrows, axis=0)

        t2 = _dot(emb_ref[:], Aws_ref[0])
        tr64 = t2[0:1, :] + cf * (t2[1:2, :] - t2[0:1, :])
        er64 = emb_ref[0:1, :] + cf * (emb_ref[1:2, :] - emb_ref[0:1, :])
        tr_ref[...] = jnp.concatenate([tr64, tr64], axis=1)
        er_ref[...] = jnp.concatenate([er64, er64], axis=1)

        zq = jnp.zeros((H, H), dtype=bf16)
        for c, wref in enumerate((Us_ref, Vs_ref, Aws_ref, Bws_ref, Cws_ref)):
            for l in range(L):
                w = wref[l].astype(bf16)
                wbd_ref[c * L + l] = jnp.concatenate(
                    [jnp.concatenate([w, zq], axis=1),
                     jnp.concatenate([zq, w], axis=1)], axis=0)

    lane = jax.lax.broadcasted_iota(jnp.int32, (1, H2), 1)
    sgn = jnp.where(lane < H, 1.0, -1.0).astype(f32)

    def ln2(t):
        q = t * t
        sall = jnp.sum(t, axis=-1, keepdims=True)
        sdiff = jnp.sum(t * sgn, axis=-1, keepdims=True)
        qall = jnp.sum(q, axis=-1, keepdims=True)
        qdiff = jnp.sum(q * sgn, axis=-1, keepdims=True)
        m = (sall + sgn * sdiff) * (0.5 / H)
        va = (qall + sgn * qdiff) * (0.5 / H) - m * m
        return (t - m) * jax.lax.rsqrt(va + 1e-5)

    def dup(a):
        return jnp.concatenate([a, a], axis=1)

    R_redT = rt_ref[...]
    P = p_ref[...]
    Pu = pu_ref[...]
    Pv = pv_ref[...]
    Smf = sm_ref[...]
    tr = tr_ref[...]
    er = er_ref[...]
    e0 = dup(emb_ref[0:1, :])
    e1 = dup(emb_ref[1:2, :])
    g0 = jax.nn.sigmoid(e0)
    g1 = jax.nn.sigmoid(e1)
    deg = jnp.sum(Smf, axis=1, keepdims=True)

    def Ubd(l):
        return wbd_ref[0 * L + l]

    def Vbd(l):
        return wbd_ref[1 * L + l]

    def Abd(l):
        return wbd_ref[2 * L + l]

    def Bbd(l):
        return wbd_ref[3 * L + l]

    def Cbd(l):
        return wbd_ref[4 * L + l]

    def rep_u(a):
        return jnp.broadcast_to(a[:, None, :], (V, VP, H2)).reshape(VVP, H2)

    def tile_v(a):
        ap = jnp.concatenate(
            [a, jnp.zeros((VP - V, H2), dtype=a.dtype)], axis=0)
        return jnp.broadcast_to(ap[None, :, :], (V, VP, H2)).reshape(VVP, H2)

    for p in range(BB // 2):
        xp = jnp.concatenate([x_ref[2 * p], x_ref[2 * p + 1]], axis=1)

        Vh = _bdot(xp, Vbd(0))
        SV = _dot(Smf, Vh)
        sumV = jnp.sum(Vh, axis=0, keepdims=True)
        num = g0 * (sumV - SV) + g1 * SV
        den = g0 * (float(V) - deg) + g1 * deg
        agg = num / (den + 1e-6)
        xp = xp + jax.nn.relu(ln2(_bdot(xp, Ubd(0)) + agg))

        bx = _bdot(xp, Bbd(0))
        cx = _bdot(xp, Cbd(0))
        s = er + jax.nn.relu(ln2(tr + rep_u(bx) + tile_v(cx)))

        for l in range(1, L):
            g = jax.nn.sigmoid(s).astype(bf16)
            Vh = _bdot(xp, Vbd(l)).astype(bf16)
            num = _dot(R_redT, g * tile_v(Vh))
            den = _dot(R_redT, g)
            agg = num / (den + 1e-6)
            xp = xp + jax.nn.relu(ln2(_bdot(xp, Ubd(l)) + agg))
            bx = _bdot(xp, Bbd(l))
            cx = _bdot(xp, Cbd(l))
            if l < L - 1:
                s = s + jax.nn.relu(ln2(_bdot(s, Abd(l)) + rep_u(bx)
                                        + tile_v(cx)))
            else:
                s_sw = _bdot(P, s)
                y = _bdot(s_sw, Abd(l)) + _dot(Pu, bx) + _dot(Pv, cx)
                res = s_sw + jax.nn.relu(ln2(y))
                sw_out_ref[2 * p] = res[:, :H]
                sw_out_ref[2 * p + 1] = res[:, H:]

        x_out_ref[2 * p] = xp[:, :H]
        x_out_ref[2 * p + 1] = xp[:, H:]


def _head_body(mlp_ref, W1_ref, b1_ref, W2_ref, b2_ref, efc_ref, etc_ref,
               efr_ref, etr_ref, z_ref, zc_ref, *, B, V, M, NSW):
    f32 = jnp.float32
    h = jax.nn.relu(_dot(mlp_ref[:], W1_ref[:]) + b1_ref[:])
    out = _dot(h, W2_ref[:]) + b2_ref[:]

    psw = jax.nn.sigmoid(out[:, M + V:M + V + NSW])
    topo = jnp.concatenate(
        [jnp.ones((B, M - NSW), dtype=f32), psw], axis=1)
    vv = jnp.concatenate(
        [jnp.ones((B, 1), dtype=f32), out[:, M + 1:M + V]], axis=1)
    pfc = out[:, :M] * topo

    nio = jax.lax.broadcasted_iota(jnp.int32, (V, M), 0)
    G = (efr_ref[:] == nio).astype(f32) - (etr_ref[:] == nio).astype(f32)
    qfc = _dot(vv, G) * topo

    eio = jax.lax.broadcasted_iota(jnp.int32, (M, V), 1)
    E = (etc_ref[:] == eio).astype(f32) - (efc_ref[:] == eio).astype(f32)
    pg = _dot(pfc, E)
    qg = _dot(qfc, E)

    z_ref[:] = jnp.concatenate([pfc, vv, topo], axis=1)
    zc_ref[:] = jnp.concatenate([qfc, pg, qg], axis=1)


def kernel(x_mod, ei, si, embed_table, Us, Vs, Aws, Bws, Cws, W1, b1, W2, b2):
    B, V, H = x_mod.shape
    L = Us.shape[0]
    NSW = si.shape[1]
    NE = ei.shape[1]
    M = NE + NSW
    BB = _BB if B % _BB == 0 else 2
    VP = (V + 7) // 8 * 8

    iu = jnp.minimum(si[0], si[1])
    ju = jnp.maximum(si[0], si[1])
    order = jnp.argsort(iu * V + ju)
    iu = iu[order]
    ju = ju[order]
    sw_sorted = jnp.stack([iu, ju])
    ef = jnp.concatenate([ei[0], iu])
    et = jnp.concatenate([ei[1], ju])

    gnn = pl.pallas_call(
        functools.partial(_gnn_body, V=V, H=H, L=L, NSW=NSW, BB=BB),
        grid=(B // BB,),
        in_specs=[
            pl.BlockSpec(memory_space=pltpu.SMEM),
            pl.BlockSpec(memory_space=pltpu.SMEM),
            pl.BlockSpec((BB, V, H), lambda i: (i, 0, 0)),
            pl.BlockSpec((2, H), lambda i: (0, 0)),
            pl.BlockSpec((L, H, H), lambda i: (0, 0, 0)),
            pl.BlockSpec((L, H, H), lambda i: (0, 0, 0)),
            pl.BlockSpec((L, H, H), lambda i: (0, 0, 0)),
            pl.BlockSpec((L, H, H), lambda i: (0, 0, 0)),
            pl.BlockSpec((L, H, H), lambda i: (0, 0, 0)),
        ],
        out_specs=[
            pl.BlockSpec((BB, V, H), lambda i: (i, 0, 0)),
            pl.BlockSpec((BB, NSW, H), lambda i: (i, 0, 0)),
        ],
        out_shape=[
            jax.ShapeDtypeStruct((B, V, H), jnp.float32),
            jax.ShapeDtypeStruct((B, NSW, H), jnp.float32),
        ],
        scratch_shapes=[
            pltpu.VMEM((V, V * VP), jnp.bfloat16),
            pltpu.VMEM((NSW, V * VP), jnp.bfloat16),
            pltpu.VMEM((NSW, V), jnp.float32),
            pltpu.VMEM((NSW, V), jnp.float32),
            pltpu.VMEM((V, V), jnp.float32),
            pltpu.VMEM((V * VP, 2 * H), jnp.float32),
            pltpu.VMEM((V * VP, 2 * H), jnp.float32),
            pltpu.VMEM((5 * L, 2 * H, 2 * H), jnp.bfloat16),
        ],
        compiler_params=pltpu.CompilerParams(
            dimension_semantics=("arbitrary",)),
    )
    x3, sw = gnn(si, sw_sorted, x_mod, embed_table, Us, Vs, Aws, Bws, Cws)

    mlp_in = jnp.concatenate(
        [sw.reshape(B, NSW * H), x3.reshape(B, V * H)], axis=1)

    head = pl.pallas_call(
        functools.partial(_head_body, B=B, V=V, M=M, NSW=NSW),
        out_shape=[
            jax.ShapeDtypeStruct((B, M + V + M), jnp.float32),
            jax.ShapeDtypeStruct((B, M + V + V), jnp.float32),
        ],
    )
    z, zc = head(mlp_in, W1, b1.reshape(1, -1), W2, b2.reshape(1, -1),
                 ef.reshape(M, 1), et.reshape(M, 1),
                 ef.reshape(1, M), et.reshape(1, M))
    return (z, zc)


# speedup vs baseline: 1.0868x; 1.0868x over previous
import functools

import jax
import jax.numpy as jnp
from jax.experimental import pallas as pl
from jax.experimental.pallas import tpu as pltpu

_BB = 8


def _dot(a, b):
    return jnp.dot(a, b, preferred_element_type=jnp.float32)


def _bdot(a, b):
    return jnp.dot(a.astype(jnp.bfloat16), b.astype(jnp.bfloat16),
                   preferred_element_type=jnp.float32)


def _gnn_body(si_ref, sw_ref, x_ref, emb_ref, Us_ref, Vs_ref, Aws_ref,
              Bws_ref, Cws_ref, x_out_ref, sw_out_ref,
              rt_ref, p_ref, pu_ref, pv_ref, sm_ref, tr_ref, er_ref, wbd_ref,
              *, V, H, L, NSW, BB):
    VP = (V + 7) // 8 * 8
    VVP = V * VP
    H2 = 2 * H
    f32 = jnp.float32
    bf16 = jnp.bfloat16

    @pl.when(pl.program_id(0) == 0)
    def _precompute():
        r_iota = jax.lax.broadcasted_iota(jnp.int32, (VVP, 1), 0)
        u_idx = r_iota // VP
        v_idx = r_iota - u_idx * VP
        valid = v_idx < V

        rrow = jax.lax.broadcasted_iota(jnp.int32, (V, VVP), 0)
        rcol = jax.lax.broadcasted_iota(jnp.int32, (V, VVP), 1)
        rt_ref[...] = ((rrow == rcol // VP) & (rcol % VP < V)).astype(bf16)

        a0 = jax.lax.broadcasted_iota(jnp.int32, (V, V), 0)
        a1 = jax.lax.broadcasted_iota(jnp.int32, (V, V), 1)
        cls = jnp.zeros((VVP, 1), dtype=jnp.bool_)
        Sm = jnp.zeros((V, V), dtype=jnp.bool_)
        for k in range(NSW):
            i0 = si_ref[0, k]
            i1 = si_ref[1, k]
            cls = cls | ((u_idx == i0) & (v_idx == i1)) | ((u_idx == i1) & (v_idx == i0))
            Sm = Sm | ((a0 == i0) & (a1 == i1)) | ((a0 == i1) & (a1 == i0))
        cf = (cls & valid).astype(f32)
        sm_ref[...] = Sm.astype(f32)

        c1 = jax.lax.broadcasted_iota(jnp.int32, (1, VVP), 1)
        cv = jax.lax.broadcasted_iota(jnp.int32, (1, V), 1)
        prows, urows, vrows = [], [], []
        for k in range(NSW):
            iu = sw_ref[0, k]
            ju = sw_ref[1, k]
            prows.append((c1 == iu * VP + ju).astype(f32))
            urows.append((cv == iu).astype(f32))
            vrows.append((cv == ju).astype(f32))
        p_ref[...] = jnp.concatenate(prows, axis=0).astype(bf16)
        pu_ref[...] = jnp.concatenate(urows, axis=0)
        pv_ref[...] = jnp.concatenate(vrows, axis=0)

        t2 = _dot(emb_ref[:], Aws_ref[0])
        tr64 = t2[0:1, :] + cf * (t2[1:2, :] - t2[0:1, :])
        er64 = emb_ref[0:1, :] + cf * (emb_ref[1:2, :] - emb_ref[0:1, :])
        tr_ref[...] = jnp.concatenate([tr64, tr64], axis=1)
        er_ref[...] = jnp.concatenate([er64, er64], axis=1)

        zq = jnp.zeros((H, H), dtype=bf16)
        for c, wref in enumerate((Us_ref, Vs_ref, Aws_ref, Bws_ref, Cws_ref)):
            for l in range(L):
                w = wref[l].astype(bf16)
                wbd_ref[c * L + l] = jnp.concatenate(
                    [jnp.concatenate([w, zq], axis=1),
                     jnp.concatenate([zq, w], axis=1)], axis=0)

    lane = jax.lax.broadcasted_iota(jnp.int32, (1, H2), 1)
    sgn = jnp.where(lane < H, 1.0, -1.0).astype(f32)

    def ln2(t):
        q = t * t
        sall = jnp.sum(t, axis=-1, keepdims=True)
        sdiff = jnp.sum(t * sgn, axis=-1, keepdims=True)
        qall = jnp.sum(q, axis=-1, keepdims=True)
        qdiff = jnp.sum(q * sgn, axis=-1, keepdims=True)
        m = (sall + sgn * sdiff) * (0.5 / H)
        va = (qall + sgn * qdiff) * (0.5 / H) - m * m
        return (t - m) * jax.lax.rsqrt(va + 1e-5)

    def dup(a):
        return jnp.concatenate([a, a], axis=1)

    R_redT = rt_ref[...]
    P = p_ref[...]
    Pu = pu_ref[...]
    Pv = pv_ref[...]
    Smf = sm_ref[...]
    tr = tr_ref[...]
    er = er_ref[...]
    e0 = dup(emb_ref[0:1, :])
    e1 = dup(emb_ref[1:2, :])
    g0 = jax.nn.sigmoid(e0)
    g1 = jax.nn.sigmoid(e1)
    deg = jnp.sum(Smf, axis=1, keepdims=True)

    def Ubd(l):
        return wbd_ref[0 * L + l]

    def Vbd(l):
        return wbd_ref[1 * L + l]

    def Abd(l):
        return wbd_ref[2 * L + l]

    def Bbd(l):
        return wbd_ref[3 * L + l]

    def Cbd(l):
        return wbd_ref[4 * L + l]

    def rep_u(a):
        return jnp.broadcast_to(a[:, None, :], (V, VP, H2)).reshape(VVP, H2)

    def tile_v(a):
        ap = jnp.concatenate(
            [a, jnp.zeros((VP - V, H2), dtype=a.dtype)], axis=0)
        return jnp.broadcast_to(ap[None, :, :], (V, VP, H2)).reshape(VVP, H2)

    for p in range(BB // 2):
        xp = jnp.concatenate([x_ref[2 * p], x_ref[2 * p + 1]], axis=1)

        Vh = _bdot(xp, Vbd(0))
        SV = _dot(Smf, Vh)
        sumV = jnp.sum(Vh, axis=0, keepdims=True)
        num = g0 * (sumV - SV) + g1 * SV
        den = g0 * (float(V) - deg) + g1 * deg
        agg = num / (den + 1e-6)
        xp = xp + jax.nn.relu(ln2(_bdot(xp, Ubd(0)) + agg))

        bx = _bdot(xp, Bbd(0))
        cx = _bdot(xp, Cbd(0))
        s = er + jax.nn.relu(ln2(tr + rep_u(bx) + tile_v(cx)))

        for l in range(1, L):
            g = jax.nn.sigmoid(s).astype(bf16)
            Vh = _bdot(xp, Vbd(l)).astype(bf16)
            num = _dot(R_redT, g * tile_v(Vh))
            den = _dot(R_redT, g)
            agg = num / (den + 1e-6)
            xp = xp + jax.nn.relu(ln2(_bdot(xp, Ubd(l)) + agg))
            bx = _bdot(xp, Bbd(l))
            cx = _bdot(xp, Cbd(l))
            if l < L - 1:
                s = s + jax.nn.relu(ln2(_bdot(s, Abd(l)) + rep_u(bx)
                                        + tile_v(cx)))
            else:
                s_sw = _bdot(P, s)
                y = _bdot(s_sw, Abd(l)) + _dot(Pu, bx) + _dot(Pv, cx)
                res = s_sw + jax.nn.relu(ln2(y))
                sw_out_ref[2 * p] = res[:, :H]
                sw_out_ref[2 * p + 1] = res[:, H:]

        x_out_ref[2 * p] = xp[:, :H]
        x_out_ref[2 * p + 1] = xp[:, H:]


def _head_body(mlp_ref, W1_ref, b1_ref, W2_ref, b2_ref, efc_ref, etc_ref,
               efr_ref, etr_ref, z_ref, zc_ref, *, B, V, M, NSW):
    f32 = jnp.float32
    h = jax.nn.relu(_dot(mlp_ref[:], W1_ref[:]) + b1_ref[:])
    out = _dot(h, W2_ref[:]) + b2_ref[:]

    psw = jax.nn.sigmoid(out[:, M + V:M + V + NSW])
    topo = jnp.concatenate(
        [jnp.ones((B, M - NSW), dtype=f32), psw], axis=1)
    vv = jnp.concatenate(
        [jnp.ones((B, 1), dtype=f32), out[:, M + 1:M + V]], axis=1)
    pfc = out[:, :M] * topo

    nio = jax.lax.broadcasted_iota(jnp.int32, (V, M), 0)
    G = (efr_ref[:] == nio).astype(f32) - (etr_ref[:] == nio).astype(f32)
    qfc = _dot(vv, G) * topo

    eio = jax.lax.broadcasted_iota(jnp.int32, (M, V), 1)
    E = (etc_ref[:] == eio).astype(f32) - (efc_ref[:] == eio).astype(f32)
    pg = _dot(pfc, E)
    qg = _dot(qfc, E)

    z_ref[:] = jnp.concatenate([pfc, vv, topo], axis=1)
    zc_ref[:] = jnp.concatenate([qfc, pg, qg], axis=1)


def kernel(x_mod, ei, si, embed_table, Us, Vs, Aws, Bws, Cws, W1, b1, W2, b2):
    B, V, H = x_mod.shape
    L = Us.shape[0]
    NSW = si.shape[1]
    NE = ei.shape[1]
    M = NE + NSW
    BB = _BB if B % _BB == 0 else 2
    VP = (V + 7) // 8 * 8

    iu = jnp.minimum(si[0], si[1])
    ju = jnp.maximum(si[0], si[1])
    order = jnp.argsort(iu * V + ju)
    iu = iu[order]
    ju = ju[order]
    sw_sorted = jnp.stack([iu, ju])
    ef = jnp.concatenate([ei[0], iu])
    et = jnp.concatenate([ei[1], ju])

    gnn = pl.pallas_call(
        functools.partial(_gnn_body, V=V, H=H, L=L, NSW=NSW, BB=BB),
        grid=(B // BB,),
        in_specs=[
            pl.BlockSpec(memory_space=pltpu.SMEM),
            pl.BlockSpec(memory_space=pltpu.SMEM),
            pl.BlockSpec((BB, V, H), lambda i: (i, 0, 0)),
            pl.BlockSpec((2, H), lambda i: (0, 0)),
            pl.BlockSpec((L, H, H), lambda i: (0, 0, 0)),
            pl.BlockSpec((L, H, H), lambda i: (0, 0, 0)),
            pl.BlockSpec((L, H, H), lambda i: (0, 0, 0)),
            pl.BlockSpec((L, H, H), lambda i: (0, 0, 0)),
            pl.BlockSpec((L, H, H), lambda i: (0, 0, 0)),
        ],
        out_specs=[
            pl.BlockSpec((BB, V, H), lambda i: (i, 0, 0)),
            pl.BlockSpec((BB, NSW, H), lambda i: (i, 0, 0)),
        ],
        out_shape=[
            jax.ShapeDtypeStruct((B, V, H), jnp.float32),
            jax.ShapeDtypeStruct((B, NSW, H), jnp.float32),
        ],
        scratch_shapes=[
            pltpu.VMEM((V, V * VP), jnp.bfloat16),
            pltpu.VMEM((NSW, V * VP), jnp.bfloat16),
            pltpu.VMEM((NSW, V), jnp.float32),
            pltpu.VMEM((NSW, V), jnp.float32),
            pltpu.VMEM((V, V), jnp.float32),
            pltpu.VMEM((V * VP, 2 * H), jnp.float32),
            pltpu.VMEM((V * VP, 2 * H), jnp.float32),
            pltpu.VMEM((5 * L, 2 * H, 2 * H), jnp.bfloat16),
        ],
        compiler_params=pltpu.CompilerParams(
            dimension_semantics=("arbitrary",)),
    )
    x3, sw = gnn(si, sw_sorted, x_mod, embed_table, Us, Vs, Aws, Bws, Cws)

    mlp_in = jnp.concatenate(
        [sw.reshape(B, NSW * H), x3.reshape(B, V * H)], axis=1)

    head = pl.pallas_call(
        functools.partial(_head_body, B=B, V=V, M=M, NSW=NSW),
        out_shape=[
            jax.ShapeDtypeStruct((B, M + V + M), jnp.float32),
            jax.ShapeDtypeStruct((B, M + V + V), jnp.float32),
        ],
    )
    z, zc = head(mlp_in, W1, b1.reshape(1, -1), W2, b2.reshape(1, -1),
                 ef.reshape(M, 1), et.reshape(M, 1),
                 ef.reshape(1, M), et.reshape(1, M))
    return (z, zc)
